# baseline (speedup 1.0000x reference)
import jax
import jax.numpy as jnp
from jax import lax
from jax.experimental import pallas as pl
from jax.experimental.pallas import tpu as pltpu
from jax.experimental.pallas import tpu_sc as plsc

N_NODES_C = 10000
N_EDGES_C = 320000
D_FEAT_C = 128

NW = 32
EPW = N_EDGES_C // NW
B = 40
NB = EPW // B
ROWS_PW = 624
LANES = 16
NCOL = D_FEAT_C // LANES


def _sc_body(x_hbm, w_hbm, ii_hbm, ij_hbm, p0_hbm, p1_hbm,
             acc, iiv, ijv, xr, wj, pr, sem_ii, sem_ij, sem_in, sem_out):
    cid = lax.axis_index("c")
    sid = lax.axis_index("s")
    wid = cid * 16 + sid

    @pl.loop(0, B)
    def _zero(r):
        for c in range(NCOL):
            xr[0][r, pl.ds(c * LANES, LANES)] = jnp.zeros((LANES,), jnp.float32)

    row0 = sid * ROWS_PW
    for k in range(ROWS_PW // B):
        pltpu.sync_copy(xr[0], acc.at[pl.ds(row0 + k * B, B)])
    rem = ROWS_PW % B
    if rem:
        pltpu.sync_copy(xr[0].at[pl.ds(0, rem)],
                        acc.at[pl.ds(row0 + (ROWS_PW // B) * B, rem)])
    tail0 = 16 * ROWS_PW
    ntail = N_NODES_C - tail0

    @pl.when(sid == 0)
    def _zt():
        pltpu.sync_copy(xr[0].at[pl.ds(0, ntail)], acc.at[pl.ds(tail0, ntail)])

    plsc.subcore_barrier()

    def issue_ij(g, q):
        pltpu.make_async_copy(ij_hbm.at[wid, g], ijv[q], sem_ij[q]).start()

    def wait_ij(q):
        pltpu.make_async_copy(ij_hbm.at[wid, 0], ijv[q], sem_ij[q]).wait()

    def issue_ii(g, q):
        pltpu.make_async_copy(ii_hbm.at[wid, g], iiv[q], sem_ii[q]).start()

    def wait_ii(q):
        pltpu.make_async_copy(ii_hbm.at[wid, 0], iiv[q], sem_ii[q]).wait()

    def issue_in(g, d, q):
        pltpu.make_async_copy(x_hbm.at[ijv[q].at[0]], xr[d], sem_in[d]).start()
        pltpu.make_async_copy(
            w_hbm.at[pl.ds((wid * NB + g) * B, B)], wj[d], sem_in[d]).start()

    def wait_in(d, q):
        pltpu.make_async_copy(x_hbm.at[ijv[q].at[0]], xr[d], sem_in[d]).wait()
        pltpu.make_async_copy(x_hbm.at[pl.ds(0, B)], wj[d], sem_in[d]).wait()

    def wait_out(d, q):
        pltpu.make_async_copy(pr[d], acc.at[iiv[q].at[0]], sem_out[d]).wait()

    def compute(d):
        @plsc.parallel_loop(0, B, 1, unroll=8)
        def _mul(r):
            for c in range(NCOL):
                sl = pl.ds(c * LANES, LANES)
                pr[d][r, sl] = xr[d][r, sl] * wj[d][r, sl]

    def issue_scatter(d, q):
        pltpu.make_async_copy(
            pr[d], acc.at[iiv[q].at[0]], sem_out[d]).start(add=True)

    def step(g, gi, w_out=True, i_ii=True, i_next=True, i_ij=True):
        d, q = gi % 2, gi % 4
        if w_out:
            wait_out(d, (gi + 2) % 4)
        if i_ii:
            issue_ii(g + 2, (gi + 2) % 4)
        if i_next:
            wait_ij((gi + 1) % 4)
            issue_in(g + 1, 1 - d, (gi + 1) % 4)
        wait_in(d, q)
        if i_ij:
            issue_ij(g + 4, q)
        wait_ii(q)
        compute(d)
        issue_scatter(d, q)

    for q in range(4):
        issue_ij(jnp.int32(q), q)
    issue_ii(jnp.int32(0), 0)
    issue_ii(jnp.int32(1), 1)
    wait_ij(0)
    issue_in(jnp.int32(0), 0, 0)

    for gi in range(4):
        step(jnp.int32(gi), gi, w_out=gi >= 2)

    @pl.loop(1, 61)
    def _main(u):
        for k in range(4):
            step(u * 4 + k, k)

    for g in range(244, 248):
        step(jnp.int32(g), g, i_ij=g + 4 < NB)

    step(jnp.int32(248), 248, i_ii=False, i_ij=False)
    step(jnp.int32(249), 249, i_ii=False, i_next=False, i_ij=False)
    wait_out(0, 0)
    wait_out(1, 1)

    plsc.subcore_barrier()

    @pl.when(cid == 0)
    def _():
        pltpu.sync_copy(acc.at[pl.ds(row0, ROWS_PW)],
                        p0_hbm.at[pl.ds(row0, ROWS_PW)])

        @pl.when(sid == 0)
        def _():
            pltpu.sync_copy(acc.at[pl.ds(tail0, ntail)],
                            p0_hbm.at[pl.ds(tail0, ntail)])

    @pl.when(cid == 1)
    def _():
        pltpu.sync_copy(acc.at[pl.ds(row0, ROWS_PW)],
                        p1_hbm.at[pl.ds(row0, ROWS_PW)])

        @pl.when(sid == 0)
        def _():
            pltpu.sync_copy(acc.at[pl.ds(tail0, ntail)],
                            p1_hbm.at[pl.ds(tail0, ntail)])


_sc_conv = pl.kernel(
    _sc_body,
    out_type=(jax.ShapeDtypeStruct((N_NODES_C, D_FEAT_C), jnp.float32),
              jax.ShapeDtypeStruct((N_NODES_C, D_FEAT_C), jnp.float32)),
    mesh=plsc.VectorSubcoreMesh(core_axis_name="c", subcore_axis_name="s"),
    scratch_types=[
        pltpu.VMEM_SHARED((N_NODES_C, D_FEAT_C), jnp.float32),
        [pltpu.VMEM((1, B), jnp.int32) for _ in range(4)],
        [pltpu.VMEM((1, B), jnp.int32) for _ in range(4)],
        [pltpu.VMEM((B, D_FEAT_C), jnp.float32) for _ in range(2)],
        [pltpu.VMEM((B, D_FEAT_C), jnp.float32) for _ in range(2)],
        [pltpu.VMEM((B, D_FEAT_C), jnp.float32) for _ in range(2)],
        [pltpu.SemaphoreType.DMA for _ in range(4)],
        [pltpu.SemaphoreType.DMA for _ in range(4)],
        [pltpu.SemaphoreType.DMA for _ in range(2)],
        [pltpu.SemaphoreType.DMA for _ in range(2)],
    ],
)


def _add_body(a_ref, b_ref, o_ref):
    o_ref[...] = a_ref[...] + b_ref[...]


_combine = pl.pallas_call(
    _add_body,
    grid=(10,),
    in_specs=[pl.BlockSpec((1000, D_FEAT_C), lambda i: (i, 0))] * 2,
    out_specs=pl.BlockSpec((1000, D_FEAT_C), lambda i: (i, 0)),
    out_shape=jax.ShapeDtypeStruct((N_NODES_C, D_FEAT_C), jnp.float32),
)


@jax.jit
def kernel(x, Wij, idx_i, idx_j):
    ii = idx_i.astype(jnp.int32).reshape(NW, NB, 1, B)
    ij = idx_j.astype(jnp.int32).reshape(NW, NB, 1, B)
    p0, p1 = _sc_conv(x, Wij, ii, ij)
    return _combine(p0, p1)


# speedup vs baseline: 1.0102x; 1.0102x over previous
import jax
import jax.numpy as jnp
from jax import lax
from jax.experimental import pallas as pl
from jax.experimental.pallas import tpu as pltpu
from jax.experimental.pallas import tpu_sc as plsc

N_NODES_C = 10000
N_EDGES_C = 320000
D_FEAT_C = 128

NW = 32
EPW = N_EDGES_C // NW
B = 40
NB = EPW // B
ROWS_PW = 624
LANES = 16
NCOL = D_FEAT_C // LANES


def _sc_body(x_hbm, w_hbm, ii_hbm, ij_hbm, p0_hbm, p1_hbm,
             acc, iiv, ijv, xr, wj, pr, sem_ii, sem_ij, sem_in, sem_out):
    cid = lax.axis_index("c")
    sid = lax.axis_index("s")
    wid = cid * 16 + sid

    @pl.loop(0, B)
    def _zero(r):
        for c in range(NCOL):
            xr[0][r, pl.ds(c * LANES, LANES)] = jnp.zeros((LANES,), jnp.float32)

    row0 = sid * ROWS_PW
    for k in range(ROWS_PW // B):
        pltpu.sync_copy(xr[0], acc.at[pl.ds(row0 + k * B, B)])
    rem = ROWS_PW % B
    if rem:
        pltpu.sync_copy(xr[0].at[pl.ds(0, rem)],
                        acc.at[pl.ds(row0 + (ROWS_PW // B) * B, rem)])
    tail0 = 16 * ROWS_PW
    ntail = N_NODES_C - tail0

    @pl.when(sid == 0)
    def _zt():
        pltpu.sync_copy(xr[0].at[pl.ds(0, ntail)], acc.at[pl.ds(tail0, ntail)])

    plsc.subcore_barrier()

    def issue_ij(g, q):
        pltpu.make_async_copy(ij_hbm.at[wid, g], ijv[q], sem_ij[q]).start()

    def wait_ij(q):
        pltpu.make_async_copy(ij_hbm.at[wid, 0], ijv[q], sem_ij[q]).wait()

    def issue_ii(g, q):
        pltpu.make_async_copy(ii_hbm.at[wid, g], iiv[q], sem_ii[q]).start()

    def wait_ii(q):
        pltpu.make_async_copy(ii_hbm.at[wid, 0], iiv[q], sem_ii[q]).wait()

    def issue_in(g, d, q):
        pltpu.make_async_copy(
            w_hbm.at[pl.ds((wid * NB + g) * B, B)], wj[d], sem_in[d]).start()
        pltpu.make_async_copy(x_hbm.at[ijv[q].at[0]], xr[d], sem_in[d]).start()

    def wait_in(d, q):
        pltpu.make_async_copy(x_hbm.at[ijv[q].at[0]], xr[d], sem_in[d]).wait()
        pltpu.make_async_copy(x_hbm.at[pl.ds(0, B)], wj[d], sem_in[d]).wait()

    def wait_out(d, q):
        pltpu.make_async_copy(pr[d], acc.at[iiv[q].at[0]], sem_out[d]).wait()

    def compute(d):
        @plsc.parallel_loop(0, B, 1, unroll=4)
        def _mul(r):
            for c in range(NCOL):
                sl = pl.ds(c * LANES, LANES)
                pr[d][r, sl] = xr[d][r, sl] * wj[d][r, sl]

    def issue_scatter(d, q):
        pltpu.make_async_copy(
            pr[d], acc.at[iiv[q].at[0]], sem_out[d]).start(add=True)

    def step(g, gi, w_out=True, i_ii=True, i_next=True, i_ij=True):
        d, q = gi % 2, gi % 4
        if w_out:
            wait_out(d, (gi + 2) % 4)
        if i_ii:
            issue_ii(g + 2, (gi + 2) % 4)
        if i_next:
            pltpu.make_async_copy(
                w_hbm.at[pl.ds((wid * NB + g + 1) * B, B)],
                wj[1 - d], sem_in[1 - d]).start()
            wait_ij((gi + 1) % 4)
            pltpu.make_async_copy(
                x_hbm.at[ijv[(gi + 1) % 4].at[0]], xr[1 - d],
                sem_in[1 - d]).start()
        wait_in(d, q)
        if i_ij:
            issue_ij(g + 4, q)
        compute(d)
        wait_ii(q)
        issue_scatter(d, q)

    for q in range(4):
        issue_ij(jnp.int32(q), q)
    issue_ii(jnp.int32(0), 0)
    issue_ii(jnp.int32(1), 1)
    pltpu.make_async_copy(
        w_hbm.at[pl.ds((wid * NB) * B, B)], wj[0], sem_in[0]).start()
    wait_ij(0)
    pltpu.make_async_copy(x_hbm.at[ijv[0].at[0]], xr[0], sem_in[0]).start()

    for gi in range(4):
        step(jnp.int32(gi), gi, w_out=gi >= 2)

    @pl.loop(1, 61)
    def _main(u):
        for k in range(4):
            step(u * 4 + k, k)

    for g in range(244, 248):
        step(jnp.int32(g), g, i_ij=g + 4 < NB)

    step(jnp.int32(248), 248, i_ii=False, i_ij=False)
    step(jnp.int32(249), 249, i_ii=False, i_next=False, i_ij=False)
    wait_out(0, 0)
    wait_out(1, 1)

    plsc.subcore_barrier()

    @pl.when(cid == 0)
    def _():
        pltpu.sync_copy(acc.at[pl.ds(row0, ROWS_PW)],
                        p0_hbm.at[pl.ds(row0, ROWS_PW)])

        @pl.when(sid == 0)
        def _():
            pltpu.sync_copy(acc.at[pl.ds(tail0, ntail)],
                            p0_hbm.at[pl.ds(tail0, ntail)])

    @pl.when(cid == 1)
    def _():
        pltpu.sync_copy(acc.at[pl.ds(row0, ROWS_PW)],
                        p1_hbm.at[pl.ds(row0, ROWS_PW)])

        @pl.when(sid == 0)
        def _():
            pltpu.sync_copy(acc.at[pl.ds(tail0, ntail)],
                            p1_hbm.at[pl.ds(tail0, ntail)])


_sc_conv = pl.kernel(
    _sc_body,
    out_type=(jax.ShapeDtypeStruct((N_NODES_C, D_FEAT_C), jnp.float32),
              jax.ShapeDtypeStruct((N_NODES_C, D_FEAT_C), jnp.float32)),
    mesh=plsc.VectorSubcoreMesh(core_axis_name="c", subcore_axis_name="s"),
    scratch_types=[
        pltpu.VMEM_SHARED((N_NODES_C, D_FEAT_C), jnp.float32),
        [pltpu.VMEM((1, B), jnp.int32) for _ in range(4)],
        [pltpu.VMEM((1, B), jnp.int32) for _ in range(4)],
        [pltpu.VMEM((B, D_FEAT_C), jnp.float32) for _ in range(2)],
        [pltpu.VMEM((B, D_FEAT_C), jnp.float32) for _ in range(2)],
        [pltpu.VMEM((B, D_FEAT_C), jnp.float32) for _ in range(2)],
        [pltpu.SemaphoreType.DMA for _ in range(4)],
        [pltpu.SemaphoreType.DMA for _ in range(4)],
        [pltpu.SemaphoreType.DMA for _ in range(2)],
        [pltpu.SemaphoreType.DMA for _ in range(2)],
    ],
)


def _add_body(a_ref, b_ref, o_ref):
    o_ref[...] = a_ref[...] + b_ref[...]


_combine = pl.pallas_call(
    _add_body,
    grid=(10,),
    in_specs=[pl.BlockSpec((1000, D_FEAT_C), lambda i: (i, 0))] * 2,
    out_specs=pl.BlockSpec((1000, D_FEAT_C), lambda i: (i, 0)),
    out_shape=jax.ShapeDtypeStruct((N_NODES_C, D_FEAT_C), jnp.float32),
)


@jax.jit
def kernel(x, Wij, idx_i, idx_j):
    ii = idx_i.astype(jnp.int32).reshape(NW, NB, 1, B)
    ij = idx_j.astype(jnp.int32).reshape(NW, NB, 1, B)
    p0, p1 = _sc_conv(x, Wij, ii, ij)
    return _combine(p0, p1)


# speedup vs baseline: 1.0120x; 1.0018x over previous
import jax
import jax.numpy as jnp
from jax import lax
from jax.experimental import pallas as pl
from jax.experimental.pallas import tpu as pltpu
from jax.experimental.pallas import tpu_sc as plsc

N_NODES_C = 10000
N_EDGES_C = 320000
D_FEAT_C = 128

NW = 32
EPW = N_EDGES_C // NW
B = 40
NB = EPW // B
ROWS_PW = 624
LANES = 16
NCOL = D_FEAT_C // LANES


def _sc_body(x_hbm, w_hbm, ii_hbm, ij_hbm, p0_hbm, p1_hbm,
             acc, iiv, ijv, xr, wj, pr, sem_ii, sem_ij, sem_in, sem_out):
    cid = lax.axis_index("c")
    sid = lax.axis_index("s")
    wid = cid * 16 + sid

    @pl.loop(0, B)
    def _zero(r):
        for c in range(NCOL):
            xr[0][r, pl.ds(c * LANES, LANES)] = jnp.zeros((LANES,), jnp.float32)

    row0 = sid * ROWS_PW
    for k in range(ROWS_PW // B):
        pltpu.sync_copy(xr[0], acc.at[pl.ds(row0 + k * B, B)])
    rem = ROWS_PW % B
    if rem:
        pltpu.sync_copy(xr[0].at[pl.ds(0, rem)],
                        acc.at[pl.ds(row0 + (ROWS_PW // B) * B, rem)])
    tail0 = 16 * ROWS_PW
    ntail = N_NODES_C - tail0

    @pl.when(sid == 0)
    def _zt():
        pltpu.sync_copy(xr[0].at[pl.ds(0, ntail)], acc.at[pl.ds(tail0, ntail)])

    plsc.subcore_barrier()

    def issue_ij(g, q):
        pltpu.make_async_copy(ij_hbm.at[wid, g], ijv[q], sem_ij[q]).start()

    def wait_ij(q):
        pltpu.make_async_copy(ij_hbm.at[wid, 0], ijv[q], sem_ij[q]).wait()

    def issue_ii(g, q):
        pltpu.make_async_copy(ii_hbm.at[wid, g], iiv[q], sem_ii[q]).start()

    def wait_ii(q):
        pltpu.make_async_copy(ii_hbm.at[wid, 0], iiv[q], sem_ii[q]).wait()

    def issue_in(g, d, q):
        pltpu.make_async_copy(x_hbm.at[ijv[q].at[0]], xr[d], sem_in[d]).start()
        pltpu.make_async_copy(
            w_hbm.at[pl.ds((wid * NB + g) * B, B)], wj[d], sem_in[d]).start()

    def wait_in(d, q):
        pltpu.make_async_copy(x_hbm.at[ijv[q].at[0]], xr[d], sem_in[d]).wait()
        pltpu.make_async_copy(x_hbm.at[pl.ds(0, B)], wj[d], sem_in[d]).wait()

    def wait_out(d, q):
        pltpu.make_async_copy(pr[d], acc.at[iiv[q].at[0]], sem_out[d]).wait()

    def compute(d):
        @plsc.parallel_loop(0, B, 1, unroll=4)
        def _mul(r):
            for c in range(NCOL):
                sl = pl.ds(c * LANES, LANES)
                pr[d][r, sl] = xr[d][r, sl] * wj[d][r, sl]

    def issue_scatter(d, q):
        pltpu.make_async_copy(
            pr[d], acc.at[iiv[q].at[0]], sem_out[d]).start(add=True)

    def step(g, gi, w_out=True, i_ii=True, i_next=True, i_ij=True):
        d, q = gi % 2, gi % 4
        if w_out:
            wait_out(d, (gi + 2) % 4)
        if i_ii:
            issue_ii(g + 2, (gi + 2) % 4)
        if i_next:
            wait_ij((gi + 1) % 4)
            issue_in(g + 1, 1 - d, (gi + 1) % 4)
        wait_in(d, q)
        if i_ij:
            issue_ij(g + 4, q)
        compute(d)
        wait_ii(q)
        issue_scatter(d, q)

    for q in range(4):
        issue_ij(jnp.int32(q), q)
    issue_ii(jnp.int32(0), 0)
    issue_ii(jnp.int32(1), 1)
    wait_ij(0)
    issue_in(jnp.int32(0), 0, 0)

    for gi in range(4):
        step(jnp.int32(gi), gi, w_out=gi >= 2)

    @pl.loop(1, 61)
    def _main(u):
        for k in range(4):
            step(u * 4 + k, k)

    for g in range(244, 248):
        step(jnp.int32(g), g, i_ij=g + 4 < NB)

    step(jnp.int32(248), 248, i_ii=False, i_ij=False)
    step(jnp.int32(249), 249, i_ii=False, i_next=False, i_ij=False)
    wait_out(0, 0)
    wait_out(1, 1)

    plsc.subcore_barrier()

    @pl.when(cid == 0)
    def _():
        pltpu.sync_copy(acc.at[pl.ds(row0, ROWS_PW)],
                        p0_hbm.at[pl.ds(row0, ROWS_PW)])

        @pl.when(sid == 0)
        def _():
            pltpu.sync_copy(acc.at[pl.ds(tail0, ntail)],
                            p0_hbm.at[pl.ds(tail0, ntail)])

    @pl.when(cid == 1)
    def _():
        pltpu.sync_copy(acc.at[pl.ds(row0, ROWS_PW)],
                        p1_hbm.at[pl.ds(row0, ROWS_PW)])

        @pl.when(sid == 0)
        def _():
            pltpu.sync_copy(acc.at[pl.ds(tail0, ntail)],
                            p1_hbm.at[pl.ds(tail0, ntail)])


_sc_conv = pl.kernel(
    _sc_body,
    out_type=(jax.ShapeDtypeStruct((N_NODES_C, D_FEAT_C), jnp.float32),
              jax.ShapeDtypeStruct((N_NODES_C, D_FEAT_C), jnp.float32)),
    mesh=plsc.VectorSubcoreMesh(core_axis_name="c", subcore_axis_name="s"),
    scratch_types=[
        pltpu.VMEM_SHARED((N_NODES_C, D_FEAT_C), jnp.float32),
        [pltpu.VMEM((1, B), jnp.int32) for _ in range(4)],
        [pltpu.VMEM((1, B), jnp.int32) for _ in range(4)],
        [pltpu.VMEM((B, D_FEAT_C), jnp.float32) for _ in range(2)],
        [pltpu.VMEM((B, D_FEAT_C), jnp.float32) for _ in range(2)],
        [pltpu.VMEM((B, D_FEAT_C), jnp.float32) for _ in range(2)],
        [pltpu.SemaphoreType.DMA for _ in range(4)],
        [pltpu.SemaphoreType.DMA for _ in range(4)],
        [pltpu.SemaphoreType.DMA for _ in range(2)],
        [pltpu.SemaphoreType.DMA for _ in range(2)],
    ],
)


def _add_body(a_ref, b_ref, o_ref):
    o_ref[...] = a_ref[...] + b_ref[...]


_combine = pl.pallas_call(
    _add_body,
    grid=(10,),
    in_specs=[pl.BlockSpec((1000, D_FEAT_C), lambda i: (i, 0))] * 2,
    out_specs=pl.BlockSpec((1000, D_FEAT_C), lambda i: (i, 0)),
    out_shape=jax.ShapeDtypeStruct((N_NODES_C, D_FEAT_C), jnp.float32),
)


@jax.jit
def kernel(x, Wij, idx_i, idx_j):
    ii = idx_i.astype(jnp.int32).reshape(NW, NB, 1, B)
    ij = idx_j.astype(jnp.int32).reshape(NW, NB, 1, B)
    p0, p1 = _sc_conv(x, Wij, ii, ij)
    return _combine(p0, p1)
